# baseline (speedup 1.0000x reference)
import jax
import jax.numpy as jnp
from jax import lax
from jax.experimental import pallas as pl

NUM_CODE = 8192
CODE_DIM = 32
ROW_BLOCK = 512
CODE_CHUNK = 2048
N_CHUNKS = NUM_CODE // CODE_CHUNK


def _vq_kernel(z_ref, zn_ref, cb_ref, cn_ref, zq_ref, idx_ref):
    z = z_ref[...]
    zn = zn_ref[...]
    z16 = z.astype(jnp.bfloat16)

    acc_v = None
    acc_i = None
    for c in range(N_CHUNKS):
        cbc = cb_ref[pl.ds(c * CODE_CHUNK, CODE_CHUNK), :]
        cnc = cn_ref[0:1, pl.ds(c * CODE_CHUNK, CODE_CHUNK)]
        dot = lax.dot_general(z16, cbc, (((1,), (1,)), ((), ())),
                              preferred_element_type=jnp.float32)
        d = (zn + cnc) - 2.0 * dot
        mnc = jnp.min(d, axis=1, keepdims=True)
        iot = lax.broadcasted_iota(jnp.int32, d.shape, 1) + c * CODE_CHUNK
        idxc = jnp.min(jnp.where(d == mnc, iot, jnp.int32(2**30)),
                       axis=1, keepdims=True)
        if c == 0:
            acc_v, acc_i = mnc, idxc
        else:
            accq = acc_v.astype(jnp.bfloat16).astype(jnp.float32)
            keep = accq <= mnc
            acc_v = jnp.where(keep, acc_v, mnc)
            acc_i = jnp.where(keep, acc_i, idxc)

    q = jnp.zeros((ROW_BLOCK, CODE_DIM), dtype=jnp.float32)
    for c in range(N_CHUNKS):
        cbc = cb_ref[pl.ds(c * CODE_CHUNK, CODE_CHUNK), :]
        iot = lax.broadcasted_iota(jnp.int32, (ROW_BLOCK, CODE_CHUNK), 1)
        oh = (acc_i == iot + c * CODE_CHUNK).astype(jnp.float32)
        q = q + lax.dot_general(oh, cbc, (((1,), (0,)), ((), ())),
                                preferred_element_type=jnp.float32,
                                precision=lax.Precision.HIGHEST)

    zq_ref[...] = z + (q - z)
    idx_ref[...] = acc_i


def kernel(z_e, codebook):
    n, dch, h, w = z_e.shape
    z = jnp.transpose(z_e, (0, 2, 3, 1))
    z_flat = z.reshape(-1, CODE_DIM)
    b = z_flat.shape[0]
    zn = jnp.sum(z_flat ** 2, axis=1, keepdims=True)
    cn = jnp.sum(codebook ** 2, axis=1).reshape(1, NUM_CODE)

    nb = b // ROW_BLOCK
    zq_flat, idx = pl.pallas_call(
        _vq_kernel,
        grid=(nb,),
        in_specs=[
            pl.BlockSpec((ROW_BLOCK, CODE_DIM), lambda i: (i, 0)),
            pl.BlockSpec((ROW_BLOCK, 1), lambda i: (i, 0)),
            pl.BlockSpec((NUM_CODE, CODE_DIM), lambda i: (0, 0)),
            pl.BlockSpec((1, NUM_CODE), lambda i: (0, 0)),
        ],
        out_specs=[
            pl.BlockSpec((ROW_BLOCK, CODE_DIM), lambda i: (i, 0)),
            pl.BlockSpec((ROW_BLOCK, 1), lambda i: (i, 0)),
        ],
        out_shape=[
            jax.ShapeDtypeStruct((b, CODE_DIM), jnp.float32),
            jax.ShapeDtypeStruct((b, 1), jnp.int32),
        ],
    )(z_flat, zn, codebook, cn)

    z_q = jnp.transpose(zq_flat.reshape(n, h, w, dch), (0, 3, 1, 2))
    indices = idx.reshape(n, h, w)
    return (z_q, indices)


# speedup vs baseline: 1.8275x; 1.8275x over previous
import jax
import jax.numpy as jnp
from jax import lax
from jax.experimental import pallas as pl

NUM_CODE = 8192
CODE_DIM = 32
ROW_BLOCK = 512
CODE_CHUNK = 2048
N_CHUNKS = NUM_CODE // CODE_CHUNK


def _vq_kernel(z_ref, zn_ref, cb_ref, cn_ref, zq_ref, idx_ref):
    z = z_ref[...]
    zn = zn_ref[...]
    z16 = z.astype(jnp.bfloat16)

    acc_v = None
    acc_i = None
    for c in range(N_CHUNKS):
        cbc = cb_ref[pl.ds(c * CODE_CHUNK, CODE_CHUNK), :]
        cnc = cn_ref[0:1, pl.ds(c * CODE_CHUNK, CODE_CHUNK)]
        dot = lax.dot_general(z16, cbc, (((1,), (1,)), ((), ())),
                              preferred_element_type=jnp.float32)
        d = (zn + cnc) - 2.0 * dot
        mnc = jnp.min(d, axis=1, keepdims=True)
        iot = lax.broadcasted_iota(jnp.int32, d.shape, 1) + c * CODE_CHUNK
        idxc = jnp.min(jnp.where(d == mnc, iot, jnp.int32(2**30)),
                       axis=1, keepdims=True)
        if c == 0:
            acc_v, acc_i = mnc, idxc
        else:
            accq = acc_v.astype(jnp.bfloat16).astype(jnp.float32)
            keep = accq <= mnc
            acc_v = jnp.where(keep, acc_v, mnc)
            acc_i = jnp.where(keep, acc_i, idxc)

    q = jnp.zeros((ROW_BLOCK, CODE_DIM), dtype=jnp.float32)
    for c in range(N_CHUNKS):
        cbc = cb_ref[pl.ds(c * CODE_CHUNK, CODE_CHUNK), :]
        iot = lax.broadcasted_iota(jnp.int32, (ROW_BLOCK, CODE_CHUNK), 1)
        oh = (acc_i == iot + c * CODE_CHUNK).astype(jnp.bfloat16)
        q = q + lax.dot_general(oh, cbc, (((1,), (0,)), ((), ())),
                                preferred_element_type=jnp.float32)

    zq_ref[...] = z + (q - z)
    idx_ref[...] = acc_i


def kernel(z_e, codebook):
    n, dch, h, w = z_e.shape
    z = jnp.transpose(z_e, (0, 2, 3, 1))
    z_flat = z.reshape(-1, CODE_DIM)
    b = z_flat.shape[0]
    zn = jnp.sum(z_flat ** 2, axis=1, keepdims=True)
    cn = jnp.sum(codebook ** 2, axis=1).reshape(1, NUM_CODE)

    nb = b // ROW_BLOCK
    zq_flat, idx = pl.pallas_call(
        _vq_kernel,
        grid=(nb,),
        in_specs=[
            pl.BlockSpec((ROW_BLOCK, CODE_DIM), lambda i: (i, 0)),
            pl.BlockSpec((ROW_BLOCK, 1), lambda i: (i, 0)),
            pl.BlockSpec((NUM_CODE, CODE_DIM), lambda i: (0, 0)),
            pl.BlockSpec((1, NUM_CODE), lambda i: (0, 0)),
        ],
        out_specs=[
            pl.BlockSpec((ROW_BLOCK, CODE_DIM), lambda i: (i, 0)),
            pl.BlockSpec((ROW_BLOCK, 1), lambda i: (i, 0)),
        ],
        out_shape=[
            jax.ShapeDtypeStruct((b, CODE_DIM), jnp.float32),
            jax.ShapeDtypeStruct((b, 1), jnp.int32),
        ],
    )(z_flat, zn, codebook, cn)

    z_q = jnp.transpose(zq_flat.reshape(n, h, w, dch), (0, 3, 1, 2))
    indices = idx.reshape(n, h, w)
    return (z_q, indices)


# speedup vs baseline: 2.0630x; 1.1289x over previous
import jax
import jax.numpy as jnp
from jax import lax
from jax.experimental import pallas as pl

NUM_CODE = 8192
CODE_DIM = 32
ROW_BLOCK = 512
CODE_CHUNK = 2048
N_CHUNKS = NUM_CODE // CODE_CHUNK


def _vq_kernel(z_ref, zn_ref, cb_ref, cn_ref, zq_ref, idx_ref):
    z = z_ref[...]
    zn = zn_ref[...]
    z16 = z.astype(jnp.bfloat16)

    acc_v = None
    acc_i = None
    for c in range(N_CHUNKS):
        cbc = cb_ref[pl.ds(c * CODE_CHUNK, CODE_CHUNK), :]
        cnc = cn_ref[0:1, pl.ds(c * CODE_CHUNK, CODE_CHUNK)]
        dot = lax.dot_general(z16, cbc, (((1,), (1,)), ((), ())),
                              preferred_element_type=jnp.float32)
        d = (zn + cnc) - 2.0 * dot
        mnc = jnp.min(d, axis=1, keepdims=True)
        idxc = (jnp.argmin(d, axis=1).astype(jnp.int32)
                + c * CODE_CHUNK).reshape(ROW_BLOCK, 1)
        if c == 0:
            acc_v, acc_i = mnc, idxc
        else:
            accq = acc_v.astype(jnp.bfloat16).astype(jnp.float32)
            keep = accq <= mnc
            acc_v = jnp.where(keep, acc_v, mnc)
            acc_i = jnp.where(keep, acc_i, idxc)

    q = jnp.zeros((ROW_BLOCK, CODE_DIM), dtype=jnp.float32)
    for c in range(N_CHUNKS):
        cbc = cb_ref[pl.ds(c * CODE_CHUNK, CODE_CHUNK), :]
        iot = lax.broadcasted_iota(jnp.int32, (ROW_BLOCK, CODE_CHUNK), 1)
        oh = (acc_i == iot + c * CODE_CHUNK).astype(jnp.bfloat16)
        q = q + lax.dot_general(oh, cbc, (((1,), (0,)), ((), ())),
                                preferred_element_type=jnp.float32)

    zq_ref[...] = z + (q - z)
    idx_ref[...] = acc_i


def kernel(z_e, codebook):
    n, dch, h, w = z_e.shape
    z = jnp.transpose(z_e, (0, 2, 3, 1))
    z_flat = z.reshape(-1, CODE_DIM)
    b = z_flat.shape[0]
    zn = jnp.sum(z_flat ** 2, axis=1, keepdims=True)
    cn = jnp.sum(codebook ** 2, axis=1).reshape(1, NUM_CODE)

    nb = b // ROW_BLOCK
    zq_flat, idx = pl.pallas_call(
        _vq_kernel,
        grid=(nb,),
        in_specs=[
            pl.BlockSpec((ROW_BLOCK, CODE_DIM), lambda i: (i, 0)),
            pl.BlockSpec((ROW_BLOCK, 1), lambda i: (i, 0)),
            pl.BlockSpec((NUM_CODE, CODE_DIM), lambda i: (0, 0)),
            pl.BlockSpec((1, NUM_CODE), lambda i: (0, 0)),
        ],
        out_specs=[
            pl.BlockSpec((ROW_BLOCK, CODE_DIM), lambda i: (i, 0)),
            pl.BlockSpec((ROW_BLOCK, 1), lambda i: (i, 0)),
        ],
        out_shape=[
            jax.ShapeDtypeStruct((b, CODE_DIM), jnp.float32),
            jax.ShapeDtypeStruct((b, 1), jnp.int32),
        ],
    )(z_flat, zn, codebook, cn)

    z_q = jnp.transpose(zq_flat.reshape(n, h, w, dch), (0, 3, 1, 2))
    indices = idx.reshape(n, h, w)
    return (z_q, indices)
